# baseline (speedup 1.0000x reference)
import functools

import jax
import jax.numpy as jnp
from jax import lax
from jax.experimental import pallas as pl
from jax.experimental.pallas import tpu as pltpu
from jax.experimental.pallas import tpu_sc as plsc

SIGMA = 2.0
NC = 2
NS = 16
NW = NC * NS
GCH = 200


def _sc_gather(feats, pts16, idx_flat):
    B = idx_flat.shape[0]
    C = feats.shape[1]
    b_per_w = B // NW
    niter = b_per_w // GCH
    assert niter % 2 == 0 and niter * GCH == b_per_w
    mesh = plsc.VectorSubcoreMesh(core_axis_name="c", subcore_axis_name="s")

    @functools.partial(
        pl.kernel,
        mesh=mesh,
        compiler_params=pltpu.CompilerParams(use_tc_tiling_on_sc=False),
        out_type=[
            jax.ShapeDtypeStruct((B, C), feats.dtype),
            jax.ShapeDtypeStruct((B, 16), pts16.dtype),
        ],
        scratch_types=[
            pltpu.VMEM((b_per_w,), jnp.int32),
            pltpu.VMEM((GCH, C), feats.dtype),
            pltpu.VMEM((GCH, C), feats.dtype),
            pltpu.VMEM((GCH, 16), pts16.dtype),
            pltpu.VMEM((GCH, 16), pts16.dtype),
            pltpu.SemaphoreType.DMA,
            pltpu.SemaphoreType.DMA,
            pltpu.SemaphoreType.DMA,
            pltpu.SemaphoreType.DMA,
        ],
    )
    def gather_kernel(feats_hbm, pts_hbm, idx_hbm, gout, pout,
                      idx_all, r0, r1, p0, p1, gs0, gs1, ws0, ws1):
        wid = lax.axis_index("s") * NC + lax.axis_index("c")
        base0 = wid * b_per_w
        pltpu.sync_copy(idx_hbm.at[pl.ds(base0, b_per_w)], idx_all)

        def start_gather(i, rows_v, pts_v, gsem):
            ix = idx_all.at[pl.ds(i * GCH, GCH)]
            cf = pltpu.async_copy(feats_hbm.at[ix], rows_v, gsem)
            cp = pltpu.async_copy(pts_hbm.at[ix], pts_v, gsem)
            return cf, cp

        def wait_writeback(rows_v, pts_v, wsem):
            pltpu.make_async_copy(rows_v, gout.at[pl.ds(base0, GCH)], wsem).wait()
            pltpu.make_async_copy(pts_v, pout.at[pl.ds(base0, GCH)], wsem).wait()

        def start_writeback(i, rows_v, pts_v, wsem):
            base = base0 + i * GCH
            pltpu.async_copy(rows_v, gout.at[pl.ds(base, GCH)], wsem)
            pltpu.async_copy(pts_v, pout.at[pl.ds(base, GCH)], wsem)

        @pl.loop(0, niter // 2)
        def _(j):
            i0 = 2 * j
            i1 = i0 + 1

            @pl.when(j > 0)
            def _():
                wait_writeback(r0, p0, ws0)

            c0f, c0p = start_gather(i0, r0, p0, gs0)

            @pl.when(j > 0)
            def _():
                wait_writeback(r1, p1, ws1)

            c1f, c1p = start_gather(i1, r1, p1, gs1)
            c0f.wait()
            c0p.wait()
            start_writeback(i0, r0, p0, ws0)
            c1f.wait()
            c1p.wait()
            start_writeback(i1, r1, p1, ws1)

        wait_writeback(r0, p0, ws0)
        wait_writeback(r1, p1, ws1)

    return gather_kernel(feats, pts16, idx_flat)


def _tc_body(H, g_ref, p_ref, q16_ref, sf_ref, wq_ref, bq_ref, wk_ref, bk_ref,
             kpmat_ref, kpsq_ref, kpw_ref, g1_ref, b1_ref, wa1_ref, g2_ref,
             b2_ref, wa2_ref, ba2_ref, tile_ref, hsum_ref, out_ref):
    f32 = jnp.float32
    g = g_ref[...]
    p = p_ref[...]
    q16 = q16_ref[...]
    sf = sf_ref[...]
    BM = q16.shape[0]
    E, C = g.shape
    CPG = wa1_ref.shape[1]
    GROUPS = C // CPG

    qe = jnp.broadcast_to(q16[:, None, :], (BM, H, 16)).reshape(E, 16)
    nbr = p - qe
    dots = jnp.dot(nbr, kpmat_ref[...], preferred_element_type=f32)
    nsq = jnp.sum(nbr * nbr, axis=-1, keepdims=True)
    sqd = nsq - 2.0 * dots + kpsq_ref[...]
    iota = lax.broadcasted_iota(jnp.int32, (E, 16), 1)
    key = lax.bitcast_convert_type(jnp.maximum(sqd, 0.0), jnp.int32)
    key = (key & jnp.int32(-16)) | iota
    kmin = jnp.min(key, axis=-1, keepdims=True)
    oh = (key == kmin).astype(f32)
    mn = lax.bitcast_convert_type(kmin & jnp.int32(-16), jnp.float32)
    infl = jnp.maximum(1.0 - jnp.sqrt(mn) / SIGMA, 0.0)
    w = jnp.dot(oh, kpw_ref[...], preferred_element_type=f32) * infl

    nk = jnp.dot(g, wk_ref[...], preferred_element_type=f32) + bk_ref[...]
    qf = jnp.dot(sf, wq_ref[...], preferred_element_type=f32) + bq_ref[...]
    qfe = jnp.broadcast_to(qf[:, None, :], (BM, H, C)).reshape(E, C)

    def leaky(x):
        return jnp.where(x >= 0, x, 0.1 * x)

    x = qfe - nk * w
    x = leaky(x * g1_ref[...] + b1_ref[...])
    t = jnp.dot(x, wa1_ref[...], preferred_element_type=f32)
    t = leaky(t * g2_ref[...] + b2_ref[...])
    t = jnp.dot(t, wa2_ref[...], preferred_element_type=f32) + ba2_ref[...]
    a = jax.nn.sigmoid(t)

    afull = jnp.dot(a, tile_ref[...], preferred_element_type=f32)
    prod = g * afull
    out_ref[...] = jnp.dot(hsum_ref[...], prod, preferred_element_type=f32)


def _tc_pass(G, P, q16, s_feats, Wq, bq, Wk, bk, kpmat, kpsq, kpw,
             g1, b1, Wa1, g2, b2, Wa2, ba2, tilemat, hsummat, BM, H,
             interpret=False):
    M = q16.shape[0]
    C = s_feats.shape[1]
    CPG = Wa1.shape[1]
    E = BM * H
    grid = (M // BM,)

    def full(shape):
        return pl.BlockSpec(shape, lambda i: (0, 0))

    return pl.pallas_call(
        functools.partial(_tc_body, H),
        grid=grid,
        in_specs=[
            pl.BlockSpec((E, C), lambda i: (i, 0)),
            pl.BlockSpec((E, 16), lambda i: (i, 0)),
            pl.BlockSpec((BM, 16), lambda i: (i, 0)),
            pl.BlockSpec((BM, C), lambda i: (i, 0)),
            full((C, C)),
            full((1, C)),
            full((C, C)),
            full((1, C)),
            full((16, 16)),
            full((1, 16)),
            full((16, C)),
            full((1, C)),
            full((1, C)),
            full((C, CPG)),
            full((1, CPG)),
            full((1, CPG)),
            full((CPG, CPG)),
            full((1, CPG)),
            full((CPG, C)),
            full((BM, E)),
        ],
        out_specs=pl.BlockSpec((BM, C), lambda i: (i, 0)),
        out_shape=jax.ShapeDtypeStruct((M, C), jnp.float32),
        interpret=interpret,
    )(G, P, q16, s_feats, Wq, bq, Wk, bk, kpmat, kpsq, kpw,
      g1, b1, Wa1, g2, b2, Wa2, ba2, tilemat, hsummat)


def kernel(q_pts, s_pts, s_feats, neighb_inds, Wq, bq, Wk, bk, kp_weights,
           bn1_g, bn1_b, Wa1, bn2_g, bn2_b, Wa2, ba2, kernel_points):
    M, H = neighb_inds.shape
    C = s_feats.shape[1]
    K = kp_weights.shape[0]

    idx = neighb_inds.reshape(-1).astype(jnp.int32)
    pts16 = jnp.concatenate(
        [s_pts, jnp.zeros((s_pts.shape[0], 13), jnp.float32)], axis=1)
    q16 = jnp.concatenate(
        [q_pts, jnp.zeros((M, 13), jnp.float32)], axis=1)

    kpmat = jnp.zeros((16, 16), jnp.float32)
    kpmat = kpmat.at[:3, :K].set(kernel_points.T)
    kpsq = jnp.full((1, 16), 1e9, jnp.float32)
    kpsq = kpsq.at[0, :K].set(jnp.sum(kernel_points * kernel_points, axis=1))
    kpw = jnp.zeros((16, C), jnp.float32).at[:K, :].set(kp_weights)

    BM = 80
    NCHUNK = 5
    MC = M // NCHUNK
    CPG = Wa1.shape[1]
    tilemat = jnp.tile(jnp.eye(CPG, dtype=jnp.float32), (1, C // CPG))
    hsummat = jnp.repeat(jnp.eye(BM, dtype=jnp.float32), H, axis=1)

    outs = []
    for c in range(NCHUNK):
        sl = slice(c * MC, (c + 1) * MC)
        G, P = _sc_gather(s_feats, pts16, idx[c * MC * H:(c + 1) * MC * H])
        outs.append(_tc_pass(
            G, P, q16[sl], s_feats[sl], Wq, bq.reshape(1, C), Wk,
            bk.reshape(1, C), kpmat, kpsq, kpw, bn1_g.reshape(1, C),
            bn1_b.reshape(1, C), Wa1, bn2_g.reshape(1, -1),
            bn2_b.reshape(1, -1), Wa2, ba2.reshape(1, -1),
            tilemat, hsummat, BM=BM, H=H))
    return jnp.concatenate(outs, axis=0)


# speedup vs baseline: 1.0564x; 1.0564x over previous
import functools

import jax
import jax.numpy as jnp
from jax import lax
from jax.experimental import pallas as pl
from jax.experimental.pallas import tpu as pltpu
from jax.experimental.pallas import tpu_sc as plsc

SIGMA = 2.0
NC = 2
NS = 16
NW = NC * NS
GCH = 200


def _sc_gather(table, idx_flat, tc_tiling):
    B = idx_flat.shape[0]
    D = table.shape[1]
    b_per_w = B // NW
    niter = b_per_w // GCH
    assert niter % 2 == 0 and niter * GCH == b_per_w
    mesh = plsc.VectorSubcoreMesh(core_axis_name="c", subcore_axis_name="s")

    @functools.partial(
        pl.kernel,
        mesh=mesh,
        compiler_params=pltpu.CompilerParams(use_tc_tiling_on_sc=tc_tiling),
        out_type=jax.ShapeDtypeStruct((B, D), table.dtype),
        scratch_types=[
            pltpu.VMEM((b_per_w,), jnp.int32),
            pltpu.VMEM((GCH, D), table.dtype),
            pltpu.VMEM((GCH, D), table.dtype),
            pltpu.SemaphoreType.DMA,
            pltpu.SemaphoreType.DMA,
            pltpu.SemaphoreType.DMA,
            pltpu.SemaphoreType.DMA,
        ],
    )
    def gather_kernel(table_hbm, idx_hbm, gout,
                      idx_all, r0, r1, gs0, gs1, ws0, ws1):
        wid = lax.axis_index("s") * NC + lax.axis_index("c")
        base0 = wid * b_per_w
        pltpu.sync_copy(idx_hbm.at[pl.ds(base0, b_per_w)], idx_all)

        def start_gather(i, rows_v, gsem):
            ix = idx_all.at[pl.ds(i * GCH, GCH)]
            return pltpu.async_copy(table_hbm.at[ix], rows_v, gsem)

        def wait_writeback(rows_v, wsem):
            pltpu.make_async_copy(rows_v, gout.at[pl.ds(base0, GCH)], wsem).wait()

        def start_writeback(i, rows_v, wsem):
            pltpu.async_copy(rows_v, gout.at[pl.ds(base0 + i * GCH, GCH)], wsem)

        @pl.loop(0, niter // 2)
        def _(j):
            i0 = 2 * j
            i1 = i0 + 1

            @pl.when(j > 0)
            def _():
                wait_writeback(r0, ws0)

            c0 = start_gather(i0, r0, gs0)

            @pl.when(j > 0)
            def _():
                wait_writeback(r1, ws1)

            c1 = start_gather(i1, r1, gs1)
            c0.wait()
            start_writeback(i0, r0, ws0)
            c1.wait()
            start_writeback(i1, r1, ws1)

        wait_writeback(r0, ws0)
        wait_writeback(r1, ws1)

    return gather_kernel(table, idx_flat)


def _tc_body(H, g_ref, p_ref, q16_ref, sf_ref, wq_ref, bq_ref, wk_ref, bk_ref,
             kpmat_ref, kpsq_ref, kpw_ref, g1_ref, b1_ref, wa1_ref, g2_ref,
             b2_ref, wa2_ref, ba2_ref, tile_ref, hsum_ref, out_ref):
    f32 = jnp.float32
    g = g_ref[...]
    p = p_ref[...]
    q16 = q16_ref[...]
    sf = sf_ref[...]
    BM = q16.shape[0]
    E, C = g.shape
    CPG = wa1_ref.shape[1]
    GROUPS = C // CPG

    qe = jnp.broadcast_to(q16[:, None, :], (BM, H, 16)).reshape(E, 16)
    nbr = p - qe
    dots = jnp.dot(nbr, kpmat_ref[...], preferred_element_type=f32)
    nsq = jnp.sum(nbr * nbr, axis=-1, keepdims=True)
    sqd = nsq - 2.0 * dots + kpsq_ref[...]
    iota = lax.broadcasted_iota(jnp.int32, (E, 16), 1)
    key = lax.bitcast_convert_type(jnp.maximum(sqd, 0.0), jnp.int32)
    key = (key & jnp.int32(-16)) | iota
    kmin = jnp.min(key, axis=-1, keepdims=True)
    oh = (key == kmin).astype(f32)
    mn = lax.bitcast_convert_type(kmin & jnp.int32(-16), jnp.float32)
    infl = jnp.maximum(1.0 - jnp.sqrt(mn) / SIGMA, 0.0)
    w = jnp.dot(oh, kpw_ref[...], preferred_element_type=f32) * infl

    nk = jnp.dot(g, wk_ref[...], preferred_element_type=f32) + bk_ref[...]
    qf = jnp.dot(sf, wq_ref[...], preferred_element_type=f32) + bq_ref[...]
    qfe = jnp.broadcast_to(qf[:, None, :], (BM, H, C)).reshape(E, C)

    def leaky(x):
        return jnp.where(x >= 0, x, 0.1 * x)

    x = qfe - nk * w
    x = leaky(x * g1_ref[...] + b1_ref[...])
    t = jnp.dot(x, wa1_ref[...], preferred_element_type=f32)
    t = leaky(t * g2_ref[...] + b2_ref[...])
    t = jnp.dot(t, wa2_ref[...], preferred_element_type=f32) + ba2_ref[...]
    a = jax.nn.sigmoid(t)

    afull = jnp.dot(a, tile_ref[...], preferred_element_type=f32)
    prod = g * afull
    out_ref[...] = jnp.dot(hsum_ref[...], prod, preferred_element_type=f32)


def _tc_pass(G, P, q16, s_feats, Wq, bq, Wk, bk, kpmat, kpsq, kpw,
             g1, b1, Wa1, g2, b2, Wa2, ba2, tilemat, hsummat, BM, H, MC,
             off, interpret=False):
    C = s_feats.shape[1]
    CPG = Wa1.shape[1]
    E = BM * H
    grid = (MC // BM,)

    def full(shape):
        return pl.BlockSpec(shape, lambda i: (0, 0))

    return pl.pallas_call(
        functools.partial(_tc_body, H),
        grid=grid,
        in_specs=[
            pl.BlockSpec((E, C), lambda i: (i, 0)),
            pl.BlockSpec((E, 16), lambda i: (i + off, 0)),
            pl.BlockSpec((BM, 16), lambda i: (i + off, 0)),
            pl.BlockSpec((BM, C), lambda i: (i + off, 0)),
            full((C, C)),
            full((1, C)),
            full((C, C)),
            full((1, C)),
            full((16, 16)),
            full((1, 16)),
            full((16, C)),
            full((1, C)),
            full((1, C)),
            full((C, CPG)),
            full((1, CPG)),
            full((1, CPG)),
            full((CPG, CPG)),
            full((1, CPG)),
            full((CPG, C)),
            full((BM, E)),
        ],
        out_specs=pl.BlockSpec((BM, C), lambda i: (i, 0)),
        out_shape=jax.ShapeDtypeStruct((MC, C), jnp.float32),
        interpret=interpret,
    )(G, P, q16, s_feats, Wq, bq, Wk, bk, kpmat, kpsq, kpw,
      g1, b1, Wa1, g2, b2, Wa2, ba2, tilemat, hsummat)


def kernel(q_pts, s_pts, s_feats, neighb_inds, Wq, bq, Wk, bk, kp_weights,
           bn1_g, bn1_b, Wa1, bn2_g, bn2_b, Wa2, ba2, kernel_points):
    M, H = neighb_inds.shape
    C = s_feats.shape[1]
    K = kp_weights.shape[0]

    idx = neighb_inds.reshape(-1).astype(jnp.int32)
    pts16 = jnp.concatenate(
        [s_pts, jnp.zeros((s_pts.shape[0], 13), jnp.float32)], axis=1)
    q16 = jnp.concatenate(
        [q_pts, jnp.zeros((M, 13), jnp.float32)], axis=1)

    kpmat = jnp.zeros((16, 16), jnp.float32)
    kpmat = kpmat.at[:3, :K].set(kernel_points.T)
    kpsq = jnp.full((1, 16), 1e9, jnp.float32)
    kpsq = kpsq.at[0, :K].set(jnp.sum(kernel_points * kernel_points, axis=1))
    kpw = jnp.zeros((16, C), jnp.float32).at[:K, :].set(kp_weights)

    BM = 80
    NCHUNK = 5
    MC = M // NCHUNK
    CPG = Wa1.shape[1]
    tilemat = jnp.tile(jnp.eye(CPG, dtype=jnp.float32), (1, C // CPG))
    hsummat = jnp.repeat(jnp.eye(BM, dtype=jnp.float32), H, axis=1)

    P = _sc_gather(pts16, idx, tc_tiling=False)
    outs = []
    for c in range(NCHUNK):
        G = _sc_gather(s_feats, idx[c * MC * H:(c + 1) * MC * H],
                       tc_tiling=True)
        outs.append(_tc_pass(
            G, P, q16, s_feats, Wq, bq.reshape(1, C), Wk,
            bk.reshape(1, C), kpmat, kpsq, kpw, bn1_g.reshape(1, C),
            bn1_b.reshape(1, C), Wa1, bn2_g.reshape(1, -1),
            bn2_b.reshape(1, -1), Wa2, ba2.reshape(1, -1),
            tilemat, hsummat, BM=BM, H=H, MC=MC, off=c * (MC // BM)))
    return jnp.concatenate(outs, axis=0)
